# baseline (speedup 1.0000x reference)
import functools

import jax
import jax.numpy as jnp
from jax import lax
from jax.experimental import pallas as pl
from jax.experimental.pallas import tpu as pltpu
from jax.experimental.pallas import tpu_sc as plsc

NUM_EXPERTS = 8
TOP_K = 2
LANES = 16
NUM_WORKERS = 32
TBLK = 1024


def _gate_logits_body(w_ref, x_ref, out_ref):
    out_ref[...] = lax.dot_general(
        w_ref[...], x_ref[...],
        dimension_numbers=(((1,), (1,)), ((), ())),
        preferred_element_type=jnp.float32)


def _gate_logits(x, w, tokens):
    hidden = x.shape[1]
    return pl.pallas_call(
        _gate_logits_body,
        grid=(tokens // TBLK,),
        in_specs=[
            pl.BlockSpec((NUM_EXPERTS, hidden), lambda i: (0, 0)),
            pl.BlockSpec((TBLK, hidden), lambda i: (i, 0)),
        ],
        out_specs=pl.BlockSpec((NUM_EXPERTS, TBLK), lambda i: (0, i)),
        out_shape=jax.ShapeDtypeStruct((NUM_EXPERTS, tokens), jnp.float32),
        compiler_params=pltpu.CompilerParams(
            dimension_semantics=("arbitrary",)),
    )(w, x)


_GATHER_DNUMS = lax.GatherDimensionNumbers(
    offset_dims=(), collapsed_slice_dims=(0,), start_index_map=(0,))


def _vgather(x, idx):
    return lax.gather(x, idx[:, None], _GATHER_DNUMS, (1,),
                      mode=lax.GatherScatterMode.PROMISE_IN_BOUNDS)


def _interleave(a, b, idx_lo, idx_hi, even):
    lo = jnp.where(even, _vgather(a, idx_lo), _vgather(b, idx_lo))
    hi = jnp.where(even, _vgather(a, idx_hi), _vgather(b, idx_hi))
    return lo, hi


def _routing_body(tok_per_w, tokens, logits_hbm, w_hbm, e_hbm, lv, wv, ev):
    wid = lax.axis_index("s") * 2 + lax.axis_index("c")
    base = wid * tok_per_w
    for e in range(NUM_EXPERTS):
        pltpu.sync_copy(logits_hbm.at[e, pl.ds(base, tok_per_w)],
                        lv.at[pl.ds(e * tok_per_w, tok_per_w)])

    neg_inf = jnp.full((LANES,), -jnp.inf, jnp.float32)
    lanes = lax.iota(jnp.int32, LANES)
    idx_lo = lanes >> 1
    idx_hi = idx_lo + (LANES // 2)
    even = (lanes & 1) == 0

    def group(g, carry):
        t0 = g * LANES
        ls = [lv[pl.ds(e * tok_per_w + t0, LANES)] for e in range(NUM_EXPERTS)]
        m1 = ls[0]
        for e in range(1, NUM_EXPERTS):
            m1 = jnp.maximum(m1, ls[e])
        e1 = jnp.zeros((LANES,), jnp.int32)
        for e in range(NUM_EXPERTS - 1, -1, -1):
            e1 = jnp.where(ls[e] == m1, jnp.full((LANES,), e, jnp.int32), e1)
        ls2 = [jnp.where(e1 == jnp.full((LANES,), e, jnp.int32), neg_inf, ls[e])
               for e in range(NUM_EXPERTS)]
        m2 = ls2[0]
        for e in range(1, NUM_EXPERTS):
            m2 = jnp.maximum(m2, ls2[e])
        e2 = jnp.zeros((LANES,), jnp.int32)
        for e in range(NUM_EXPERTS - 1, -1, -1):
            e2 = jnp.where(ls2[e] == m2, jnp.full((LANES,), e, jnp.int32), e2)
        t = jnp.exp(m2 - m1)
        w1 = 1.0 / (1.0 + t)
        w2 = t * w1
        w_lo, w_hi = _interleave(w1, w2, idx_lo, idx_hi, even)
        e_lo, e_hi = _interleave(e1, e2, idx_lo, idx_hi, even)
        o0 = g * (LANES * TOP_K)
        wv[pl.ds(o0, LANES)] = w_lo
        wv[pl.ds(o0 + LANES, LANES)] = w_hi
        ev[pl.ds(o0, LANES)] = e_lo
        ev[pl.ds(o0 + LANES, LANES)] = e_hi
        return carry

    lax.fori_loop(0, tok_per_w // LANES, group, 0)
    obase = wid * (tok_per_w * TOP_K)
    pltpu.sync_copy(wv, w_hbm.at[pl.ds(obase, tok_per_w * TOP_K)])
    pltpu.sync_copy(ev, e_hbm.at[pl.ds(obase, tok_per_w * TOP_K)])


def _routing(logits_t, tokens):
    tok_per_w = tokens // NUM_WORKERS
    mesh = plsc.VectorSubcoreMesh(core_axis_name="c", subcore_axis_name="s")
    fn = pl.kernel(
        functools.partial(_routing_body, tok_per_w, tokens),
        mesh=mesh,
        out_type=[
            jax.ShapeDtypeStruct((tokens * TOP_K,), jnp.float32),
            jax.ShapeDtypeStruct((tokens * TOP_K,), jnp.int32),
        ],
        scratch_types=[
            pltpu.VMEM((NUM_EXPERTS * tok_per_w,), jnp.float32),
            pltpu.VMEM((TOP_K * tok_per_w,), jnp.float32),
            pltpu.VMEM((TOP_K * tok_per_w,), jnp.int32),
        ],
    )
    return fn(logits_t)


def kernel(hidden_states, gate_weight):
    batch, seq, hidden = hidden_states.shape
    tokens = batch * seq
    x = hidden_states.reshape(tokens, hidden)
    logits_t = _gate_logits(x, gate_weight, tokens)
    w_flat, e_flat = _routing(logits_t, tokens)
    return (w_flat.reshape(tokens, TOP_K), e_flat.reshape(tokens, TOP_K))


# speedup vs baseline: 1.1192x; 1.1192x over previous
import functools

import jax
import jax.numpy as jnp
from jax import lax
from jax.experimental import pallas as pl
from jax.experimental.pallas import tpu as pltpu
from jax.experimental.pallas import tpu_sc as plsc

NUM_EXPERTS = 8
TOP_K = 2
LANES = 16
NUM_WORKERS = 32
TBLK = 1024


def _gate_logits_body(w_ref, x_ref, out_ref):
    out_ref[...] = lax.dot_general(
        w_ref[...], x_ref[...],
        dimension_numbers=(((1,), (1,)), ((), ())),
        preferred_element_type=jnp.float32)


def _gate_logits(x, w, tokens):
    hidden = x.shape[1]
    return pl.pallas_call(
        _gate_logits_body,
        grid=(tokens // TBLK,),
        in_specs=[
            pl.BlockSpec((NUM_EXPERTS, hidden), lambda i: (0, 0)),
            pl.BlockSpec((TBLK, hidden), lambda i: (i, 0)),
        ],
        out_specs=pl.BlockSpec((NUM_EXPERTS, TBLK), lambda i: (0, i)),
        out_shape=jax.ShapeDtypeStruct((NUM_EXPERTS, tokens), jnp.float32),
        compiler_params=pltpu.CompilerParams(
            dimension_semantics=("arbitrary",)),
    )(w, x)


_GATHER_DNUMS = lax.GatherDimensionNumbers(
    offset_dims=(), collapsed_slice_dims=(0,), start_index_map=(0,))


def _vgather(x, idx):
    return lax.gather(x, idx[:, None], _GATHER_DNUMS, (1,),
                      mode=lax.GatherScatterMode.PROMISE_IN_BOUNDS)


def _interleave(a, b, idx_lo, idx_hi, even):
    lo = jnp.where(even, _vgather(a, idx_lo), _vgather(b, idx_lo))
    hi = jnp.where(even, _vgather(a, idx_hi), _vgather(b, idx_hi))
    return lo, hi


def _routing_body(tok_per_w, tokens, logits_hbm, w_hbm, e_hbm, lv, wv, ev):
    wid = lax.axis_index("s") * 2 + lax.axis_index("c")
    base = wid * tok_per_w
    for e in range(NUM_EXPERTS):
        pltpu.sync_copy(logits_hbm.at[e, pl.ds(base, tok_per_w)],
                        lv.at[pl.ds(e * tok_per_w, tok_per_w)])

    neg_inf = jnp.full((LANES,), -jnp.inf, jnp.float32)
    lanes = lax.iota(jnp.int32, LANES)
    idx_lo = lanes >> 1
    idx_hi = idx_lo + (LANES // 2)
    even = (lanes & 1) == 0

    def group(g, carry):
        t0 = g * LANES
        ls = [lv[pl.ds(e * tok_per_w + t0, LANES)] for e in range(NUM_EXPERTS)]
        m1 = ls[0]
        for e in range(1, NUM_EXPERTS):
            m1 = jnp.maximum(m1, ls[e])
        e1 = jnp.zeros((LANES,), jnp.int32)
        for e in range(NUM_EXPERTS - 1, -1, -1):
            e1 = jnp.where(ls[e] == m1, jnp.full((LANES,), e, jnp.int32), e1)
        ls2 = [jnp.where(e1 == jnp.full((LANES,), e, jnp.int32), neg_inf, ls[e])
               for e in range(NUM_EXPERTS)]
        m2 = ls2[0]
        for e in range(1, NUM_EXPERTS):
            m2 = jnp.maximum(m2, ls2[e])
        e2 = jnp.zeros((LANES,), jnp.int32)
        for e in range(NUM_EXPERTS - 1, -1, -1):
            e2 = jnp.where(ls2[e] == m2, jnp.full((LANES,), e, jnp.int32), e2)
        t = jnp.exp(m2 - m1)
        w1 = 1.0 / (1.0 + t)
        w2 = t * w1
        w_lo, w_hi = _interleave(w1, w2, idx_lo, idx_hi, even)
        e_lo, e_hi = _interleave(e1, e2, idx_lo, idx_hi, even)
        r = g * 2
        wv[r, :] = w_lo
        wv[r + 1, :] = w_hi
        ev[r, :] = e_lo
        ev[r + 1, :] = e_hi
        return carry

    lax.fori_loop(0, tok_per_w // LANES, group, 0)
    rows = tok_per_w * TOP_K // LANES
    pltpu.sync_copy(wv, w_hbm.at[pl.ds(wid * rows, rows), :])
    pltpu.sync_copy(ev, e_hbm.at[pl.ds(wid * rows, rows), :])


def _routing(logits_t, tokens):
    tok_per_w = tokens // NUM_WORKERS
    rows = tok_per_w * TOP_K // LANES
    mesh = plsc.VectorSubcoreMesh(core_axis_name="c", subcore_axis_name="s")
    fn = pl.kernel(
        functools.partial(_routing_body, tok_per_w, tokens),
        mesh=mesh,
        out_type=[
            jax.ShapeDtypeStruct((NUM_WORKERS * rows, LANES), jnp.float32),
            jax.ShapeDtypeStruct((NUM_WORKERS * rows, LANES), jnp.int32),
        ],
        scratch_types=[
            pltpu.VMEM((NUM_EXPERTS * tok_per_w,), jnp.float32),
            pltpu.VMEM((rows, LANES), jnp.float32),
            pltpu.VMEM((rows, LANES), jnp.int32),
        ],
    )
    return fn(logits_t)


def kernel(hidden_states, gate_weight):
    batch, seq, hidden = hidden_states.shape
    tokens = batch * seq
    x = hidden_states.reshape(tokens, hidden)
    logits_t = _gate_logits(x, gate_weight, tokens)
    w4, e4 = _routing(logits_t, tokens)
    return (w4.reshape(tokens, TOP_K), e4.reshape(tokens, TOP_K))


# speedup vs baseline: 1.5971x; 1.4270x over previous
import functools

import jax
import jax.numpy as jnp
from jax import lax
from jax.experimental import pallas as pl
from jax.experimental.pallas import tpu as pltpu
from jax.experimental.pallas import tpu_sc as plsc

NUM_EXPERTS = 8
TOP_K = 2
LANES = 16
NUM_WORKERS = 32
TBLK = 1024


def _gate_logits_body(w_ref, x_ref, out_ref):
    out_ref[...] = lax.dot_general(
        w_ref[...], x_ref[...],
        dimension_numbers=(((1,), (1,)), ((), ())),
        preferred_element_type=jnp.float32)


def _gate_logits(x, w, tokens):
    hidden = x.shape[1]
    return pl.pallas_call(
        _gate_logits_body,
        grid=(tokens // TBLK,),
        in_specs=[
            pl.BlockSpec((NUM_EXPERTS, hidden), lambda i: (0, 0)),
            pl.BlockSpec((TBLK, hidden), lambda i: (i, 0)),
        ],
        out_specs=pl.BlockSpec((NUM_EXPERTS, TBLK), lambda i: (0, i)),
        out_shape=jax.ShapeDtypeStruct((NUM_EXPERTS, tokens), jnp.float32),
        compiler_params=pltpu.CompilerParams(
            dimension_semantics=("arbitrary",)),
    )(w, x)


_GATHER_DNUMS = lax.GatherDimensionNumbers(
    offset_dims=(), collapsed_slice_dims=(0,), start_index_map=(0,))


def _vgather(x, idx):
    return lax.gather(x, idx[:, None], _GATHER_DNUMS, (1,),
                      mode=lax.GatherScatterMode.PROMISE_IN_BOUNDS)


def _interleave(a, b, idx_lo, idx_hi, even):
    lo = jnp.where(even, _vgather(a, idx_lo), _vgather(b, idx_lo))
    hi = jnp.where(even, _vgather(a, idx_hi), _vgather(b, idx_hi))
    return lo, hi


def _routing_body(tok_per_w, tokens, logits_hbm, w_hbm, e_hbm, lv, wv, ev):
    wid = lax.axis_index("s") * 2 + lax.axis_index("c")
    base = wid * tok_per_w
    pltpu.sync_copy(logits_hbm.at[:, pl.ds(base, tok_per_w)], lv)

    neg_inf = jnp.full((LANES,), -jnp.inf, jnp.float32)

    def group(g, carry):
        t0 = g * LANES
        ls = [lv[e, pl.ds(t0, LANES)] for e in range(NUM_EXPERTS)]
        m1 = ls[0]
        for e in range(1, NUM_EXPERTS):
            m1 = jnp.maximum(m1, ls[e])
        e1 = jnp.zeros((LANES,), jnp.int32)
        for e in range(NUM_EXPERTS - 1, -1, -1):
            e1 = jnp.where(ls[e] == m1, jnp.full((LANES,), e, jnp.int32), e1)
        ls2 = [jnp.where(e1 == jnp.full((LANES,), e, jnp.int32), neg_inf, ls[e])
               for e in range(NUM_EXPERTS)]
        m2 = ls2[0]
        for e in range(1, NUM_EXPERTS):
            m2 = jnp.maximum(m2, ls2[e])
        e2 = jnp.zeros((LANES,), jnp.int32)
        for e in range(NUM_EXPERTS - 1, -1, -1):
            e2 = jnp.where(ls2[e] == m2, jnp.full((LANES,), e, jnp.int32), e2)
        t = jnp.exp(m2 - m1)
        w1 = 1.0 / (1.0 + t)
        w2 = t * w1
        wv[0, pl.ds(t0, LANES)] = w1
        wv[1, pl.ds(t0, LANES)] = w2
        ev[0, pl.ds(t0, LANES)] = e1
        ev[1, pl.ds(t0, LANES)] = e2
        return carry

    lax.fori_loop(0, tok_per_w // LANES, group, 0)
    for k in range(TOP_K):
        pltpu.sync_copy(wv.at[k, :], w_hbm.at[k, pl.ds(base, tok_per_w)])
        pltpu.sync_copy(ev.at[k, :], e_hbm.at[k, pl.ds(base, tok_per_w)])


def _routing(logits_t, tokens):
    tok_per_w = tokens // NUM_WORKERS
    mesh = plsc.VectorSubcoreMesh(core_axis_name="c", subcore_axis_name="s")
    fn = pl.kernel(
        functools.partial(_routing_body, tok_per_w, tokens),
        mesh=mesh,
        out_type=[
            jax.ShapeDtypeStruct((TOP_K, tokens), jnp.float32),
            jax.ShapeDtypeStruct((TOP_K, tokens), jnp.int32),
        ],
        scratch_types=[
            pltpu.VMEM((NUM_EXPERTS, tok_per_w), jnp.float32),
            pltpu.VMEM((TOP_K, tok_per_w), jnp.float32),
            pltpu.VMEM((TOP_K, tok_per_w), jnp.int32),
        ],
    )
    return fn(logits_t)


def kernel(hidden_states, gate_weight):
    batch, seq, hidden = hidden_states.shape
    tokens = batch * seq
    x = hidden_states.reshape(tokens, hidden)
    logits_t = _gate_logits(x, gate_weight, tokens)
    w_pl, e_pl = _routing(logits_t, tokens)
    return (w_pl.T, e_pl.T)


# speedup vs baseline: 1.5972x; 1.0001x over previous
import functools

import jax
import jax.numpy as jnp
from jax import lax
from jax.experimental import pallas as pl
from jax.experimental.pallas import tpu as pltpu
from jax.experimental.pallas import tpu_sc as plsc

NUM_EXPERTS = 8
TOP_K = 2
LANES = 16
NUM_WORKERS = 32
TBLK = 1024


def _gate_logits_body(w_ref, x_ref, out_ref):
    out_ref[...] = lax.dot_general(
        w_ref[...], x_ref[...],
        dimension_numbers=(((1,), (1,)), ((), ())),
        preferred_element_type=jnp.float32)


def _gate_logits(x, w, tokens):
    hidden = x.shape[1]
    return pl.pallas_call(
        _gate_logits_body,
        grid=(tokens // TBLK,),
        in_specs=[
            pl.BlockSpec((NUM_EXPERTS, hidden), lambda i: (0, 0)),
            pl.BlockSpec((TBLK, hidden), lambda i: (i, 0)),
        ],
        out_specs=pl.BlockSpec((NUM_EXPERTS, TBLK), lambda i: (0, i)),
        out_shape=jax.ShapeDtypeStruct((NUM_EXPERTS, tokens), jnp.float32),
        compiler_params=pltpu.CompilerParams(
            dimension_semantics=("arbitrary",)),
    )(w, x)


def _routing_body(tok_per_w, tokens, logits_hbm, w_hbm, e_hbm, lv, wv, ev):
    wid = lax.axis_index("s") * 2 + lax.axis_index("c")
    base = wid * tok_per_w
    pltpu.sync_copy(logits_hbm.at[:, pl.ds(base, tok_per_w)], lv)

    neg_inf = jnp.full((LANES,), -jnp.inf, jnp.float32)
    idx_c = [jnp.full((LANES,), e, jnp.int32) for e in range(NUM_EXPERTS)]

    def argmax_tree(vals, idxs):
        while len(vals) > 1:
            nv, ni = [], []
            for a in range(0, len(vals), 2):
                take = vals[a] >= vals[a + 1]
                nv.append(jnp.where(take, vals[a], vals[a + 1]))
                ni.append(jnp.where(take, idxs[a], idxs[a + 1]))
            vals, idxs = nv, ni
        return vals[0], idxs[0]

    def group(g):
        t0 = g * LANES
        ls = [lv[e, pl.ds(t0, LANES)] for e in range(NUM_EXPERTS)]
        m1, e1 = argmax_tree(ls, idx_c)
        ls2 = [jnp.where(e1 == idx_c[e], neg_inf, ls[e])
               for e in range(NUM_EXPERTS)]
        m2, e2 = argmax_tree(ls2, idx_c)
        t = jnp.exp(m2 - m1)
        w1 = 1.0 / (1.0 + t)
        w2 = t * w1
        wv[0, pl.ds(t0, LANES)] = w1
        wv[1, pl.ds(t0, LANES)] = w2
        ev[0, pl.ds(t0, LANES)] = e1
        ev[1, pl.ds(t0, LANES)] = e2

    def group2(g2, carry):
        group(g2 * 2)
        group(g2 * 2 + 1)
        return carry

    lax.fori_loop(0, tok_per_w // LANES // 2, group2, 0)
    for k in range(TOP_K):
        pltpu.sync_copy(wv.at[k, :], w_hbm.at[k, pl.ds(base, tok_per_w)])
        pltpu.sync_copy(ev.at[k, :], e_hbm.at[k, pl.ds(base, tok_per_w)])


def _routing(logits_t, tokens):
    tok_per_w = tokens // NUM_WORKERS
    mesh = plsc.VectorSubcoreMesh(core_axis_name="c", subcore_axis_name="s")
    fn = pl.kernel(
        functools.partial(_routing_body, tok_per_w, tokens),
        mesh=mesh,
        out_type=[
            jax.ShapeDtypeStruct((TOP_K, tokens), jnp.float32),
            jax.ShapeDtypeStruct((TOP_K, tokens), jnp.int32),
        ],
        scratch_types=[
            pltpu.VMEM((NUM_EXPERTS, tok_per_w), jnp.float32),
            pltpu.VMEM((TOP_K, tok_per_w), jnp.float32),
            pltpu.VMEM((TOP_K, tok_per_w), jnp.int32),
        ],
    )
    return fn(logits_t)


def kernel(hidden_states, gate_weight):
    batch, seq, hidden = hidden_states.shape
    tokens = batch * seq
    x = hidden_states.reshape(tokens, hidden)
    logits_t = _gate_logits(x, gate_weight, tokens)
    w_pl, e_pl = _routing(logits_t, tokens)
    return (w_pl.T, e_pl.T)
